# baseline (speedup 1.0000x reference)
import functools

import jax
import jax.numpy as jnp
from jax import lax
from jax.experimental import pallas as pl
from jax.experimental.pallas import tpu as pltpu
from jax.experimental.pallas import tpu_sc as plsc

LANES = 16
PADW = 128
SUB = (64, 64, 64, 8)
OFF = (0, 64, 128, 192)
NBUF = len(SUB)


def _tta_kernel(n_batch, hist, dim, num_cores, num_subcores):
    n_workers = num_cores * num_subcores
    per_w = n_batch // n_workers
    mesh = plsc.VectorSubcoreMesh(core_axis_name="c", subcore_axis_name="s")

    @functools.partial(
        pl.kernel,
        mesh=mesh,
        out_type=jax.ShapeDtypeStruct((n_batch, hist, dim), jnp.float32),
        scratch_types=(
            [pltpu.VMEM((n,), jnp.int32) for n in SUB]
            + [pltpu.VMEM((n, dim), jnp.float32) for n in SUB]
            + [pltpu.VMEM((n, PADW), jnp.float32) for n in SUB]
            + [pltpu.SemaphoreType.DMA] * (3 * NBUF)
        ),
    )
    def k(x_hbm, idx_hbm, tab_hbm, out_hbm, *scr):
        idx_v = scr[0:NBUF]
        xb_v = scr[NBUF : 2 * NBUF]
        gb_v = scr[2 * NBUF : 3 * NBUF]
        in_sem = scr[3 * NBUF : 4 * NBUF]
        g_sem = scr[4 * NBUF : 5 * NBUF]
        out_sem = scr[5 * NBUF : 6 * NBUF]

        wid = lax.axis_index("s") * num_cores + lax.axis_index("c")
        base = wid * per_w

        def fire_in(b, bi):
            pltpu.async_copy(
                idx_hbm.at[bi, pl.ds(OFF[b], SUB[b])], idx_v[b], in_sem[b]
            )
            pltpu.async_copy(
                x_hbm.at[bi, pl.ds(OFF[b], SUB[b])], xb_v[b], in_sem[b]
            )

        def wait_in(b):
            pltpu.make_async_copy(
                idx_hbm.at[0, pl.ds(0, SUB[b])], idx_v[b], in_sem[b]
            ).wait()
            pltpu.make_async_copy(
                x_hbm.at[0, pl.ds(0, SUB[b])], xb_v[b], in_sem[b]
            ).wait()

        def wait_out(b):
            pltpu.make_async_copy(
                xb_v[b], out_hbm.at[0, pl.ds(0, SUB[b])], out_sem[b]
            ).wait()

        for b in range(NBUF):
            fire_in(b, base)

        def entry_body(g, carry):
            bi = base + g
            for b in range(NBUF):
                wait_in(b)
                pltpu.async_copy(tab_hbm.at[idx_v[b]], gb_v[b], g_sem[b])
            for b in range(NBUF):
                pltpu.make_async_copy(
                    tab_hbm.at[idx_v[b]], gb_v[b], g_sem[b]
                ).wait()

                def add_rows(i, carry2, b=b):
                    r = i * 4
                    for rr in range(4):
                        for j in range(dim // LANES):
                            plsc.addupdate(
                                xb_v[b].at[r + rr, pl.ds(j * LANES, LANES)],
                                gb_v[b][r + rr, pl.ds(j * LANES, LANES)],
                            )
                    return carry2

                lax.fori_loop(0, SUB[b] // 4, add_rows, 0, unroll=4)
                pltpu.async_copy(
                    xb_v[b],
                    out_hbm.at[bi, pl.ds(OFF[b], SUB[b])],
                    out_sem[b],
                )
            for b in range(NBUF):
                wait_out(b)

                @pl.when(g < per_w - 1)
                def _():
                    fire_in(b, bi + 1)

            return carry

        lax.fori_loop(0, per_w, entry_body, 0, unroll=False)

    return k


def kernel(x, tta, embedding):
    nb, hist, d = x.shape
    vocab = embedding.shape[0]
    idx = (tta.astype(jnp.int32) - 1) % vocab
    tabp = jnp.pad(embedding, ((0, 0), (0, PADW - d)))
    info = plsc.get_sparse_core_info()
    k = _tta_kernel(nb, hist, d, info.num_cores, info.num_subcores)
    return k(x, idx, tabp)
